# baseline (speedup 1.0000x reference)
import jax
import jax.numpy as jnp
from jax.experimental import pallas as pl
from jax.experimental.pallas import tpu as pltpu

_N = 1048576
_M = 12
_MC = 4
_NQ = 7 * _M
_RAIN = 0.1
_LANES = 128
_ROWS = _N // _LANES
_G = 16
_R = _ROWS // _G
_TR = 64


def _body(xo_ref, xt_ref, mo_ref, out_ref, acc_ref):
    i = pl.program_id(0)

    @pl.when(i == 0)
    def _():
        acc_ref[...] = jnp.zeros_like(acc_ref)

    def gsum(a):
        return a.reshape(_TR // 8, 8, _LANES).sum(axis=0)

    for mc in range(0, _M, _MC):
        for rb in range(0, _R, _TR):
            sl = pl.ds(rb, _TR)
            xo = xo_ref[sl, :]
            xt = xt_ref[sl, :]
            mo = mo_ref[sl, :]
            ro = (xo >= _RAIN).astype(jnp.float32)
            rt = (xt >= _RAIN).astype(jnp.float32)
            wo = xo * ro
            wt = xt * rt
            for m in range(mc, mc + _MC):
                cf = (mo == m).astype(jnp.float32)
                p_o = cf * wo
                p_t = cf * wt
                b = m * 7
                acc_ref[b + 0] += gsum(p_o)
                acc_ref[b + 1] += gsum(p_o * wo)
                acc_ref[b + 2] += gsum(cf * ro)
                acc_ref[b + 3] += gsum(p_t)
                acc_ref[b + 4] += gsum(p_t * wt)
                acc_ref[b + 5] += gsum(cf * rt)
                acc_ref[b + 6] += gsum(cf)

    @pl.when(i == _G - 1)
    def _():
        lane = jax.lax.broadcasted_iota(jnp.int32, (1, _LANES), 1)
        row = jnp.zeros((1, _LANES), jnp.float32)
        for p in range(_NQ):
            s = jnp.sum(acc_ref[p])
            row = row + jnp.where(lane == p, s, 0.0)
        out_ref[...] = row.reshape(1, 1, _LANES)


@jax.jit
def kernel(output, target, months):
    xo = output.reshape(_ROWS, _LANES)
    xt = target.reshape(_ROWS, _LANES)
    mo = months.reshape(_ROWS, _LANES)
    partials = pl.pallas_call(
        _body,
        grid=(_G,),
        in_specs=[
            pl.BlockSpec((_R, _LANES), lambda i: (i, 0)),
            pl.BlockSpec((_R, _LANES), lambda i: (i, 0)),
            pl.BlockSpec((_R, _LANES), lambda i: (i, 0)),
        ],
        out_specs=pl.BlockSpec((1, 1, _LANES), lambda i: (0, 0, 0)),
        out_shape=jax.ShapeDtypeStruct((1, 1, _LANES), jnp.float32),
        scratch_shapes=[pltpu.VMEM((_NQ, 8, _LANES), jnp.float32)],
    )(xo, xt, mo)

    t = partials[0, 0, :_NQ].reshape(_M, 7)
    s1o, s2o, cro = t[:, 0], t[:, 1], t[:, 2]
    s1t, s2t, crt = t[:, 3], t[:, 4], t[:, 5]
    cnt = t[:, 6]
    pos = cnt > 0
    mu_o = jnp.where(pos, s1o / cnt, 0.0)
    mu_t = jnp.where(pos, s1t / cnt, 0.0)
    vo = s2o - 2.0 * mu_o * s1o + mu_o * mu_o * cro
    vt = s2t - 2.0 * mu_t * s1t + mu_t * mu_t * crt
    vo = jnp.where(pos, vo / cnt, 0.0)
    vt = jnp.where(pos, vt / cnt, 0.0)
    so = jnp.sqrt(jnp.maximum(vo, 0.0))
    st = jnp.sqrt(jnp.maximum(vt, 0.0))
    return jnp.mean((so - st) ** 2)


# speedup vs baseline: 1.0959x; 1.0959x over previous
import jax
import jax.numpy as jnp
from jax.experimental import pallas as pl
from jax.experimental.pallas import tpu as pltpu

_N = 1048576
_M = 12
_MC = 4
_NS = 6
_NQ = _NS * _M
_RAIN = 0.1
_LANES = 128
_ROWS = _N // _LANES
_G = 16
_R = _ROWS // _G
_TR = 64
_PK = 2048.0


def _body(xo_ref, xt_ref, mo_ref, out_ref, acc_ref):
    i = pl.program_id(0)

    @pl.when(i == 0)
    def _():
        acc_ref[...] = jnp.zeros_like(acc_ref)

    def gsum(a):
        return a.reshape(_TR // 8, 8, _LANES).sum(axis=0)

    for mc in range(0, _M, _MC):
        for rb in range(0, _R, _TR):
            sl = pl.ds(rb, _TR)
            xo = xo_ref[sl, :]
            xt = xt_ref[sl, :]
            mo = mo_ref[sl, :]
            ko = xo >= _RAIN
            kt = xt >= _RAIN
            wo = jnp.where(ko, xo, 0.0)
            wt = jnp.where(kt, xt, 0.0)
            rr = jnp.where(ko, 1.0, 0.0) + jnp.where(kt, _PK, 0.0)
            for m in range(mc, mc + _MC):
                cf = (mo == m).astype(jnp.float32)
                p_o = cf * wo
                p_t = cf * wt
                b = m * _NS
                acc_ref[b + 0] += gsum(p_o)
                acc_ref[b + 1] += gsum(p_o * wo)
                acc_ref[b + 2] += gsum(p_t)
                acc_ref[b + 3] += gsum(p_t * wt)
                acc_ref[b + 4] += gsum(cf * rr)
                acc_ref[b + 5] += gsum(cf)

    @pl.when(i == _G - 1)
    def _():
        lane = jax.lax.broadcasted_iota(jnp.int32, (1, _LANES), 1)
        row = jnp.zeros((1, _LANES), jnp.float32)
        for m in range(_M):
            b = m * _NS
            for q in range(4):
                row += jnp.where(lane == m * 7 + q, jnp.sum(acc_ref[b + q]), 0.0)
            crt_p = jnp.floor(acc_ref[b + 4] * (1.0 / _PK))
            cro_p = acc_ref[b + 4] - _PK * crt_p
            row += jnp.where(lane == m * 7 + 4, jnp.sum(cro_p), 0.0)
            row += jnp.where(lane == m * 7 + 5, jnp.sum(crt_p), 0.0)
            row += jnp.where(lane == m * 7 + 6, jnp.sum(acc_ref[b + 5]), 0.0)
        out_ref[...] = row.reshape(1, 1, _LANES)


@jax.jit
def kernel(output, target, months):
    xo = output.reshape(_ROWS, _LANES)
    xt = target.reshape(_ROWS, _LANES)
    mo = months.reshape(_ROWS, _LANES)
    partials = pl.pallas_call(
        _body,
        grid=(_G,),
        in_specs=[
            pl.BlockSpec((_R, _LANES), lambda i: (i, 0)),
            pl.BlockSpec((_R, _LANES), lambda i: (i, 0)),
            pl.BlockSpec((_R, _LANES), lambda i: (i, 0)),
        ],
        out_specs=pl.BlockSpec((1, 1, _LANES), lambda i: (0, 0, 0)),
        out_shape=jax.ShapeDtypeStruct((1, 1, _LANES), jnp.float32),
        scratch_shapes=[pltpu.VMEM((_NQ, 8, _LANES), jnp.float32)],
    )(xo, xt, mo)

    t = partials[0, 0, : 7 * _M].reshape(_M, 7)
    s1o, s2o = t[:, 0], t[:, 1]
    s1t, s2t = t[:, 2], t[:, 3]
    cro, crt = t[:, 4], t[:, 5]
    cnt = t[:, 6]
    pos = cnt > 0
    mu_o = jnp.where(pos, s1o / cnt, 0.0)
    mu_t = jnp.where(pos, s1t / cnt, 0.0)
    vo = s2o - 2.0 * mu_o * s1o + mu_o * mu_o * cro
    vt = s2t - 2.0 * mu_t * s1t + mu_t * mu_t * crt
    vo = jnp.where(pos, vo / cnt, 0.0)
    vt = jnp.where(pos, vt / cnt, 0.0)
    so = jnp.sqrt(jnp.maximum(vo, 0.0))
    st = jnp.sqrt(jnp.maximum(vt, 0.0))
    return jnp.mean((so - st) ** 2)


# speedup vs baseline: 1.1571x; 1.0558x over previous
import jax
import jax.numpy as jnp
from jax.experimental import pallas as pl
from jax.experimental.pallas import tpu as pltpu

_N = 1048576
_M = 12
_MC = 4
_NS = 6
_NQ = _NS * _M
_RAIN = 0.1
_LANES = 128
_ROWS = _N // _LANES
_G = 16
_R = _ROWS // _G
_TR = 64
_PK = 2048.0


def _body(xo_ref, xt_ref, mo_ref, out_ref, acc_ref):
    i = pl.program_id(0)

    @pl.when(i == 0)
    def _():
        acc_ref[...] = jnp.zeros_like(acc_ref)

    def gsum(a):
        return a.reshape(_TR // 8, 8, _LANES).sum(axis=0)

    for mc in range(0, _M, _MC):
        for rb in range(0, _R, _TR):
            sl = pl.ds(rb, _TR)
            xo = xo_ref[sl, :]
            xt = xt_ref[sl, :]
            mo = mo_ref[sl, :]
            ko = xo >= _RAIN
            kt = xt >= _RAIN
            wo = jnp.where(ko, xo, 0.0)
            wt = jnp.where(kt, xt, 0.0)
            rr = jnp.where(ko, 1.0, 0.0) + jnp.where(kt, _PK, 0.0)
            for m in range(mc, mc + _MC):
                cf = (mo == m).astype(jnp.float32)
                p_o = cf * wo
                p_t = cf * wt
                b = m * _NS
                acc_ref[b + 0] += gsum(p_o)
                acc_ref[b + 1] += gsum(p_o * wo)
                acc_ref[b + 2] += gsum(p_t)
                acc_ref[b + 3] += gsum(p_t * wt)
                acc_ref[b + 4] += gsum(cf * rr)
                acc_ref[b + 5] += gsum(cf)

    @pl.when(i == _G - 1)
    def _():
        total = jnp.float32(0.0)
        for m in range(_M):
            b = m * _NS
            s1o = jnp.sum(acc_ref[b + 0])
            s2o = jnp.sum(acc_ref[b + 1])
            s1t = jnp.sum(acc_ref[b + 2])
            s2t = jnp.sum(acc_ref[b + 3])
            crt_p = jnp.floor(acc_ref[b + 4] * (1.0 / _PK))
            cro = jnp.sum(acc_ref[b + 4] - _PK * crt_p)
            crt = jnp.sum(crt_p)
            cnt = jnp.sum(acc_ref[b + 5])
            pos = cnt > 0
            mu_o = jnp.where(pos, s1o / cnt, 0.0)
            mu_t = jnp.where(pos, s1t / cnt, 0.0)
            vo = s2o - 2.0 * mu_o * s1o + mu_o * mu_o * cro
            vt = s2t - 2.0 * mu_t * s1t + mu_t * mu_t * crt
            vo = jnp.where(pos, vo / cnt, 0.0)
            vt = jnp.where(pos, vt / cnt, 0.0)
            so = jnp.sqrt(jnp.maximum(vo, 0.0))
            st = jnp.sqrt(jnp.maximum(vt, 0.0))
            d = so - st
            total = total + d * d
        out_ref[...] = (total / _M).reshape(1, 1)


@jax.jit
def kernel(output, target, months):
    xo = output.reshape(_ROWS, _LANES)
    xt = target.reshape(_ROWS, _LANES)
    mo = months.reshape(_ROWS, _LANES)
    partials = pl.pallas_call(
        _body,
        grid=(_G,),
        in_specs=[
            pl.BlockSpec((_R, _LANES), lambda i: (i, 0)),
            pl.BlockSpec((_R, _LANES), lambda i: (i, 0)),
            pl.BlockSpec((_R, _LANES), lambda i: (i, 0)),
        ],
        out_specs=pl.BlockSpec((1, 1), lambda i: (0, 0)),
        out_shape=jax.ShapeDtypeStruct((1, 1), jnp.float32),
        scratch_shapes=[pltpu.VMEM((_NQ, 8, _LANES), jnp.float32)],
    )(xo, xt, mo)
    return partials[0, 0]
